# Initial kernel scaffold; baseline (speedup 1.0000x reference)
import jax
import jax.numpy as jnp
from jax.experimental import pallas as pl


def kernel(features, superpoint_labels, in_proj_w, in_proj_b, out_proj_w, out_proj_b, agg_w1, agg_b1, ln_g, ln_b, agg_w2, agg_b2, consistency_weight):
    raise NotImplementedError("write your pallas kernel here")


# speedup vs baseline: 4.0123x; 4.0123x over previous
import functools

import jax
import jax.numpy as jnp
import numpy as np
from jax.experimental import pallas as pl

B, N, D = 8, 4096, 256
L = 512
H = 8
HD = D // H
MAX_SP = 32
QB = 256
NB = N // QB
W = QB + 2 * MAX_SP
LC = 128

_F32 = jnp.float32


def _dot(a, b, dims):
  return jax.lax.dot_general(a, b, (dims, ((), ())),
                             preferred_element_type=_F32)


def _attn_kernel(xp_ref, xc_ref, xn_ref, lp_ref, lc_ref, ln_ref,
                 inw_ref, inb_ref, outw_ref, outb_ref, o_ref):
  qi = pl.program_id(1)
  jb = qi * QB

  xp = xp_ref[0]
  xc = xc_ref[0]
  xn = xn_ref[0]
  xw = jnp.concatenate([xp[QB - MAX_SP:], xc, xn[:MAX_SP]], axis=0)

  lp = lp_ref[0]
  lc = lc_ref[0]
  ln = ln_ref[0]
  labw = jnp.concatenate([lp[:, QB - MAX_SP:], lc, ln[:, :MAX_SP]], axis=1)
  labw_m32 = jnp.concatenate([lp[:, QB - 2 * MAX_SP:], lc], axis=1)

  jw = jb - MAX_SP + jax.lax.broadcasted_iota(jnp.int32, (1, W), 1)
  in_range = (jw >= 0) & (jw < N)
  pos_lt32 = (jw < MAX_SP) | (labw_m32 != labw)
  key_ok = in_range & pos_lt32

  lab_q = jnp.swapaxes(lc, 0, 1)
  mask = (lab_q == labw) & key_ok

  inw = inw_ref[...]
  inb = inb_ref[...]
  q = _dot(xc, inw[:D], ((1,), (1,))) + inb[:, :D]
  k = _dot(xw, inw[D:2 * D], ((1,), (1,))) + inb[:, D:2 * D]
  v = _dot(xw, inw[2 * D:], ((1,), (1,))) + inb[:, 2 * D:]

  scale = 1.0 / np.sqrt(HD)
  outs = []
  for h in range(H):
    qh = q[:, h * HD:(h + 1) * HD]
    kh = k[:, h * HD:(h + 1) * HD]
    vh = v[:, h * HD:(h + 1) * HD]
    logits = _dot(qh, kh, ((1,), (1,))) * scale
    logits = jnp.where(mask, logits, -1e9)
    m = jnp.max(logits, axis=1, keepdims=True)
    e = jnp.exp(logits - m)
    s = jnp.sum(e, axis=1, keepdims=True)
    outs.append(_dot(e / s, vh, ((1,), (0,))))
  o = jnp.concatenate(outs, axis=1)
  o = _dot(o, outw_ref[...], ((1,), (1,))) + outb_ref[...]
  o_ref[0] = o


def _segagg_kernel(o_ref, lab_ref, w1_ref, b1_ref, g_ref, beta_ref,
                   w2_ref, b2_ref, agg_ref, cnt_ref):
  o_row = o_ref[0]
  lab = lab_ref[0]
  lab_m32 = jnp.concatenate([lab[:, :MAX_SP], lab[:, :N - MAX_SP]], axis=1)
  ii = jax.lax.broadcasted_iota(jnp.int32, (1, N), 1)
  valid = ((ii < MAX_SP) | (lab_m32 != lab)).astype(_F32)

  means = []
  cnts = []
  for c in range(L // LC):
    rows = c * LC + jax.lax.broadcasted_iota(jnp.int32, (LC, 1), 0)
    oh = (lab == rows).astype(_F32)
    cnt = jnp.sum(oh, axis=1, keepdims=True)
    ssum = _dot(oh * valid, o_row, ((1,), (0,)))
    means.append(ssum / jnp.maximum(cnt, 1.0))
    cnts.append(cnt)
  mean = jnp.concatenate(means, axis=0)
  cnt_all = jnp.concatenate(cnts, axis=0)

  h = _dot(mean, w1_ref[...], ((1,), (1,))) + b1_ref[...]
  mu = jnp.mean(h, axis=1, keepdims=True)
  var = jnp.mean((h - mu) * (h - mu), axis=1, keepdims=True)
  h = (h - mu) / jnp.sqrt(var + 1e-5) * g_ref[...] + beta_ref[...]
  h = 0.5 * h * (1.0 + jax.lax.erf(h / np.sqrt(2.0)))
  agg = _dot(h, w2_ref[...], ((1,), (1,))) + b2_ref[...]

  agg_ref[0] = agg
  cnt_ref[0] = jnp.reshape(cnt_all, (1, L))


def _blend_kernel(x_ref, lab_ref, agg_ref, cnt_ref, wsig_ref, out_ref):
  x = x_ref[0]
  lab = jnp.swapaxes(lab_ref[0], 0, 1)
  li = jax.lax.broadcasted_iota(jnp.int32, (1, L), 1)
  oh = (lab == li).astype(_F32)
  gath = _dot(oh, agg_ref[0], ((1,), (0,)))
  cnt = jnp.sum(oh * cnt_ref[0], axis=1, keepdims=True)
  w = wsig_ref[0, 0]
  out_ref[0] = jnp.where(cnt >= 3.0, (1.0 - w) * x + w * gath, x)


@jax.jit
def kernel(features, superpoint_labels, in_proj_w, in_proj_b, out_proj_w,
           out_proj_b, agg_w1, agg_b1, ln_g, ln_b, agg_w2, agg_b2,
           consistency_weight):
  f32 = _F32
  labels = superpoint_labels.astype(jnp.int32)
  lab3 = labels.reshape(B * NB, 1, QB)
  inb2 = in_proj_b.reshape(1, 3 * D)
  outb2 = out_proj_b.reshape(1, D)

  full = lambda shape: pl.BlockSpec(shape, lambda b, q: (0,) * len(shape))

  o = pl.pallas_call(
      _attn_kernel,
      grid=(B, NB),
      in_specs=[
          pl.BlockSpec((1, QB, D), lambda b, q: (b, jnp.maximum(q - 1, 0), 0)),
          pl.BlockSpec((1, QB, D), lambda b, q: (b, q, 0)),
          pl.BlockSpec((1, QB, D),
                       lambda b, q: (b, jnp.minimum(q + 1, NB - 1), 0)),
          pl.BlockSpec((1, 1, QB),
                       lambda b, q: (b * NB + jnp.maximum(q - 1, 0), 0, 0)),
          pl.BlockSpec((1, 1, QB), lambda b, q: (b * NB + q, 0, 0)),
          pl.BlockSpec((1, 1, QB),
                       lambda b, q: (b * NB + jnp.minimum(q + 1, NB - 1), 0, 0)),
          full((3 * D, D)),
          full((1, 3 * D)),
          full((D, D)),
          full((1, D)),
      ],
      out_specs=pl.BlockSpec((1, QB, D), lambda b, q: (b, q, 0)),
      out_shape=jax.ShapeDtypeStruct((B, N, D), f32),
  )(features, features, features, lab3, lab3, lab3,
    in_proj_w, inb2, out_proj_w, outb2)

  full1 = lambda shape: pl.BlockSpec(shape, lambda b: (0,) * len(shape))
  agg, cnt = pl.pallas_call(
      _segagg_kernel,
      grid=(B,),
      in_specs=[
          pl.BlockSpec((1, N, D), lambda b: (b, 0, 0)),
          pl.BlockSpec((1, 1, N), lambda b: (b, 0, 0)),
          full1((D, D)),
          full1((1, D)),
          full1((1, D)),
          full1((1, D)),
          full1((D, D)),
          full1((1, D)),
      ],
      out_specs=[
          pl.BlockSpec((1, L, D), lambda b: (b, 0, 0)),
          pl.BlockSpec((1, 1, L), lambda b: (b, 0, 0)),
      ],
      out_shape=[
          jax.ShapeDtypeStruct((B, L, D), f32),
          jax.ShapeDtypeStruct((B, 1, L), f32),
      ],
  )(o, labels.reshape(B, 1, N), agg_w1, agg_b1.reshape(1, D),
    ln_g.reshape(1, D), ln_b.reshape(1, D), agg_w2, agg_b2.reshape(1, D))

  wsig = jax.nn.sigmoid(consistency_weight).reshape(1, 1).astype(f32)
  out = pl.pallas_call(
      _blend_kernel,
      grid=(B, NB),
      in_specs=[
          pl.BlockSpec((1, QB, D), lambda b, q: (b, q, 0)),
          pl.BlockSpec((1, 1, QB), lambda b, q: (b * NB + q, 0, 0)),
          pl.BlockSpec((1, L, D), lambda b, q: (b, 0, 0)),
          pl.BlockSpec((1, 1, L), lambda b, q: (b, 0, 0)),
          full((1, 1)),
      ],
      out_specs=pl.BlockSpec((1, QB, D), lambda b, q: (b, q, 0)),
      out_shape=jax.ShapeDtypeStruct((B, N, D), f32),
  )(features, lab3, agg, cnt, wsig)
  return out
